# Initial kernel scaffold; baseline (speedup 1.0000x reference)
import jax
import jax.numpy as jnp
from jax.experimental import pallas as pl


def kernel(embeddings, sp_seg, edges, weights, chunks):
    raise NotImplementedError("write your pallas kernel here")


# speedup vs baseline: 3.0143x; 3.0143x over previous
import functools

import jax
import jax.numpy as jnp
from jax import lax
from jax.experimental import pallas as pl
from jax.experimental.pallas import tpu as pltpu
from jax.experimental.pallas import tpu_sc as plsc

DELTA_VAR = 0.5
DELTA_DIST = 1.5
ALPHA = 1.0
BETA = 1.0

L = 16
NC = 2
NS = 16
D = 16
C = 128
ROWS = 144


def _rsqrt(x):
    i = plsc.bitcast(x, jnp.int32)
    i = 0x5F3759DF - (i >> 1)
    y = plsc.bitcast(i, jnp.float32)
    for _ in range(3):
        y = y * (1.5 - 0.5 * x * y * y)
    return y


def _body(emb_hbm, seg_hbm, e0_hbm, e1_hbm, w_hbm, out_hbm,
          emb_v, seg_v, tab_v, e0_v, e1_v, w_v, idxa_v, idxb_v, acc_v,
          row_v, shared):
    cid = lax.axis_index("c")
    sid = lax.axis_index("s")
    wid = cid * NS + sid

    pix = emb_v.shape[1]
    ngrp = pix // L
    ept = e0_v.shape[0]

    iota = lax.iota(jnp.int32, L)
    zeros = jnp.zeros((L,), jnp.float32)
    ones = jnp.ones((L,), jnp.float32)
    cols = [jnp.full((L,), d, jnp.int32) for d in range(D)]

    pltpu.sync_copy(emb_hbm.at[cid, :, pl.ds(sid * pix, pix)], emb_v)
    pltpu.sync_copy(seg_hbm.at[cid, pl.ds(sid * pix, pix)], seg_v)
    pltpu.sync_copy(e0_hbm.at[cid, pl.ds(sid * ept, ept)], e0_v)
    pltpu.sync_copy(e1_hbm.at[cid, pl.ds(sid * ept, ept)], e1_v)
    pltpu.sync_copy(w_hbm.at[cid, pl.ds(sid * ept, ept)], w_v)

    for r in range(ROWS):
        tab_v[r] = zeros
    for r in range(8):
        idxa_v[pl.ds(r * L, L)] = iota + r * L
    idxb_v[...] = iota + C
    acc_v[...] = zeros

    @pl.when(sid == 0)
    def _():
        pltpu.sync_copy(tab_v, shared)
    plsc.subcore_barrier()

    @pl.loop(0, ngrp)
    def _(g):
        base = g * L
        s16 = seg_v[pl.ds(base, L)]
        for d in range(D):
            plsc.addupdate_scatter(tab_v, [s16, cols[d]],
                                   emb_v[d, pl.ds(base, L)])
        plsc.addupdate_scatter(tab_v, [C + (s16 >> 4), s16 & (L - 1)], ones)

    pltpu.sync_copy(tab_v.at[pl.ds(0, C)], shared.at[idxa_v], add=True)
    pltpu.sync_copy(tab_v.at[pl.ds(C, L)], shared.at[idxb_v], add=True)
    plsc.subcore_barrier()
    pltpu.sync_copy(shared, tab_v)

    for grp in range(C // L):
        rows = iota + grp * L
        vs = [plsc.load_gather(tab_v, [rows, cols[d]]) for d in range(D)]
        nsq = vs[0] * vs[0]
        for d in range(1, D):
            nsq = nsq + vs[d] * vs[d]
        rs = _rsqrt(jnp.maximum(nsq, 1e-20))
        for d in range(D):
            plsc.store_scatter(tab_v, [rows, cols[d]], vs[d] * rs)

    @pl.loop(0, ngrp)
    def _(g):
        base = g * L
        s16 = seg_v[pl.ds(base, L)]
        dot = emb_v[0, pl.ds(base, L)] * plsc.load_gather(tab_v, [s16, cols[0]])
        for d in range(1, D):
            dot = dot + emb_v[d, pl.ds(base, L)] * plsc.load_gather(
                tab_v, [s16, cols[d]])
        n16 = plsc.load_gather(tab_v, [C + (s16 >> 4), s16 & (L - 1)])
        term = jnp.maximum((1.0 - DELTA_VAR) - dot, 0.0) / n16
        acc_v[...] = acc_v[...] + term

    inter = zeros
    for k in range(ept // L):
        a = e0_v[pl.ds(k * L, L)]
        b = e1_v[pl.ds(k * L, L)]
        dd = plsc.load_gather(tab_v, [a, cols[0]]) * plsc.load_gather(
            tab_v, [b, cols[0]])
        for d in range(1, D):
            dd = dd + plsc.load_gather(tab_v, [a, cols[d]]) * plsc.load_gather(
                tab_v, [b, cols[d]])
        wk = w_v[pl.ds(k * L, L)]
        inter = inter + jnp.maximum(DELTA_DIST - wk * (1.0 - dd), 0.0)

    maxc = jnp.full((L,), -1, jnp.int32)
    for r in range(C // L):
        cv = tab_v[C + r]
        maxc = jnp.maximum(maxc, jnp.where(cv > 0.0, iota + r * L, -1))
    c_div = jnp.broadcast_to(jnp.max(maxc) + 1, (L,)).astype(jnp.float32)

    inv_e = 1.0 / float(ept * NS)
    row_v[...] = BETA * (acc_v[...] / c_div) + (ALPHA * inv_e) * inter
    pltpu.sync_copy(row_v, out_hbm.at[wid])


@jax.jit
def _run(emb, seg, e0, e1, w):
    b, d, hw = emb.shape
    pix = hw // NS
    ept = e0.shape[1] // NS
    kern = pl.kernel(
        _body,
        out_type=jax.ShapeDtypeStruct((NC * NS, L), jnp.float32),
        mesh=plsc.VectorSubcoreMesh(core_axis_name="c", subcore_axis_name="s"),
        compiler_params=pltpu.CompilerParams(needs_layout_passes=False),
        scratch_types=[
            pltpu.VMEM((D, pix), jnp.float32),
            pltpu.VMEM((pix,), jnp.int32),
            pltpu.VMEM((ROWS, L), jnp.float32),
            pltpu.VMEM((ept,), jnp.int32),
            pltpu.VMEM((ept,), jnp.int32),
            pltpu.VMEM((ept,), jnp.float32),
            pltpu.VMEM((C,), jnp.int32),
            pltpu.VMEM((L,), jnp.int32),
            pltpu.VMEM((L,), jnp.float32),
            pltpu.VMEM((L,), jnp.float32),
            pltpu.VMEM_SHARED((ROWS, L), jnp.float32),
        ],
    )
    out = kern(emb, seg, e0, e1, w)
    return jnp.sum(out)


def kernel(embeddings, sp_seg, edges, weights, chunks=4):
    b, d, h, w = embeddings.shape
    hw = h * w
    emb = embeddings.reshape(b, d, hw)
    seg = sp_seg.reshape(b, hw).astype(jnp.int32)
    e0 = edges[:, 0, :].astype(jnp.int32)
    e1 = edges[:, 1, :].astype(jnp.int32)
    return _run(emb, seg, e0, e1, weights)


# speedup vs baseline: 3.0894x; 1.0249x over previous
import jax
import jax.numpy as jnp
from jax import lax
from jax.experimental import pallas as pl
from jax.experimental.pallas import tpu as pltpu
from jax.experimental.pallas import tpu_sc as plsc

DELTA_VAR = 0.5
DELTA_DIST = 1.5
ALPHA = 1.0
BETA = 1.0

L = 16
NC = 2
NS = 16
D = 16
C = 128
ROWS = 144


def _rsqrt(x):
    i = plsc.bitcast(x, jnp.int32)
    i = 0x5F3759DF - (i >> 1)
    y = plsc.bitcast(i, jnp.float32)
    for _ in range(3):
        y = y * (1.5 - 0.5 * x * y * y)
    return y


def _tree_sum(xs):
    xs = list(xs)
    while len(xs) > 1:
        nxt = [xs[i] + xs[i + 1] for i in range(0, len(xs) - 1, 2)]
        if len(xs) % 2:
            nxt.append(xs[-1])
        xs = nxt
    return xs[0]


def _body(embp_hbm, seg_hbm, e0_hbm, e1_hbm, w_hbm, out_hbm,
          embp_v, seg_v, cnt_v, tab_v, zero_v, e0_v, e1_v, w_v, idxb_v,
          acc_v, row_v, shared, sem_in, sem_sc):
    cid = lax.axis_index("c")
    sid = lax.axis_index("s")
    wid = cid * NS + sid

    pix = embp_v.shape[0]
    ngrp = pix // L
    ept = e0_v.shape[0]

    iota = lax.iota(jnp.int32, L)
    zeros = jnp.zeros((L,), jnp.float32)
    ones = jnp.ones((L,), jnp.float32)
    cols = [jnp.full((L,), d, jnp.int32) for d in range(D)]

    dins = [
        pltpu.make_async_copy(embp_hbm.at[cid, sid], embp_v, sem_in),
        pltpu.make_async_copy(seg_hbm.at[cid, sid], seg_v, sem_in),
        pltpu.make_async_copy(e0_hbm.at[cid, pl.ds(sid * ept, ept)], e0_v,
                              sem_in),
        pltpu.make_async_copy(e1_hbm.at[cid, pl.ds(sid * ept, ept)], e1_v,
                              sem_in),
        pltpu.make_async_copy(w_hbm.at[cid, pl.ds(sid * ept, ept)], w_v,
                              sem_in),
    ]
    for dsc in dins:
        dsc.start()

    for r in range(C):
        zero_v[r] = zeros
    for r in range(L):
        cnt_v[r] = zeros
    idxb_v[...] = iota + C
    acc_v[...] = zeros

    for dsc in dins:
        dsc.wait()

    @pl.when(sid == 0)
    def _():
        pltpu.sync_copy(zero_v, shared.at[pl.ds(0, C)])
        pltpu.sync_copy(zero_v.at[pl.ds(0, L)], shared.at[pl.ds(C, L)])
    plsc.subcore_barrier()

    dscat = [
        pltpu.make_async_copy(embp_v.at[pl.ds(j * C, C)],
                              shared.at[seg_v.at[j]], sem_sc)
        for j in range(pix // C)
    ]
    for dsc in dscat:
        dsc.start(add=True)

    @pl.loop(0, ngrp)
    def _(g):
        s16 = seg_v[g >> 3, pl.ds((g & 7) * L, L)]
        plsc.addupdate_scatter(cnt_v, [s16 >> 4, s16 & (L - 1)], ones)

    pltpu.sync_copy(cnt_v, shared.at[idxb_v], add=True)

    for dsc in dscat:
        dsc.wait()
    plsc.subcore_barrier()

    pltpu.sync_copy(shared, tab_v)
    for grp in range(C // L):
        rows = iota + grp * L
        vs = [plsc.load_gather(tab_v, [rows, cols[d]]) for d in range(D)]
        nsq = _tree_sum([v * v for v in vs])
        rs = _rsqrt(jnp.maximum(nsq, 1e-20))
        for d in range(D):
            plsc.store_scatter(tab_v, [rows, cols[d]], vs[d] * rs)

    @pl.loop(0, ngrp)
    def _(g):
        s16 = seg_v[g >> 3, pl.ds((g & 7) * L, L)]
        prows = iota + g * L
        es = [plsc.load_gather(embp_v, [prows, cols[d]]) for d in range(D)]
        ms = [plsc.load_gather(tab_v, [s16, cols[d]]) for d in range(D)]
        n16 = plsc.load_gather(tab_v, [C + (s16 >> 4), s16 & (L - 1)])
        dot = _tree_sum([e * m for e, m in zip(es, ms)])
        term = jnp.maximum((1.0 - DELTA_VAR) - dot, 0.0) / n16
        acc_v[...] = acc_v[...] + term

    inter = zeros
    for k in range(ept // L):
        a = e0_v[pl.ds(k * L, L)]
        b = e1_v[pl.ds(k * L, L)]
        mas = [plsc.load_gather(tab_v, [a, cols[d]]) for d in range(D)]
        mbs = [plsc.load_gather(tab_v, [b, cols[d]]) for d in range(D)]
        dd = _tree_sum([x * y for x, y in zip(mas, mbs)])
        wk = w_v[pl.ds(k * L, L)]
        inter = inter + jnp.maximum(DELTA_DIST - wk * (1.0 - dd), 0.0)

    maxc = jnp.full((L,), -1, jnp.int32)
    for r in range(C // L):
        cv = tab_v[C + r]
        maxc = jnp.maximum(maxc, jnp.where(cv > 0.0, iota + r * L, -1))
    c_div = jnp.broadcast_to(jnp.max(maxc) + 1, (L,)).astype(jnp.float32)

    inv_e = 1.0 / float(ept * NS)
    row_v[...] = BETA * (acc_v[...] / c_div) + (ALPHA * inv_e) * inter
    pltpu.sync_copy(row_v, out_hbm.at[wid])


@jax.jit
def _run(embp, seg, e0, e1, w):
    b, ns, pix, d = embp.shape
    ept = e0.shape[1] // NS
    kern = pl.kernel(
        _body,
        out_type=jax.ShapeDtypeStruct((NC * NS, L), jnp.float32),
        mesh=plsc.VectorSubcoreMesh(core_axis_name="c", subcore_axis_name="s"),
        compiler_params=pltpu.CompilerParams(
            needs_layout_passes=False, use_tc_tiling_on_sc=False),
        scratch_types=[
            pltpu.VMEM((pix, D), jnp.float32),
            pltpu.VMEM((pix // C, C), jnp.int32),
            pltpu.VMEM((L, L), jnp.float32),
            pltpu.VMEM((ROWS, L), jnp.float32),
            pltpu.VMEM((C, L), jnp.float32),
            pltpu.VMEM((ept,), jnp.int32),
            pltpu.VMEM((ept,), jnp.int32),
            pltpu.VMEM((ept,), jnp.float32),
            pltpu.VMEM((L,), jnp.int32),
            pltpu.VMEM((L,), jnp.float32),
            pltpu.VMEM((L,), jnp.float32),
            pltpu.VMEM_SHARED((ROWS, L), jnp.float32),
            pltpu.SemaphoreType.DMA,
            pltpu.SemaphoreType.DMA,
        ],
    )
    out = kern(embp, seg, e0, e1, w)
    return jnp.sum(out)


def kernel(embeddings, sp_seg, edges, weights, chunks=4):
    b, d, h, w = embeddings.shape
    hw = h * w
    pix = hw // NS
    embp = embeddings.reshape(b, d, hw).transpose(0, 2, 1)
    embp = embp.reshape(b, NS, pix, d)
    seg = sp_seg.reshape(b, hw).astype(jnp.int32).reshape(b, NS, pix // C, C)
    e0 = edges[:, 0, :].astype(jnp.int32)
    e1 = edges[:, 1, :].astype(jnp.int32)
    return _run(embp, seg, e0, e1, weights)


# speedup vs baseline: 4.4186x; 1.4302x over previous
import jax
import jax.numpy as jnp
from jax import lax
from jax.experimental import pallas as pl
from jax.experimental.pallas import tpu as pltpu
from jax.experimental.pallas import tpu_sc as plsc

DELTA_VAR = 0.5
DELTA_DIST = 1.5
ALPHA = 1.0
BETA = 1.0

L = 16
NC = 2
NS = 16
D = 16
C = 128
ROWS = 144


def _rsqrt(x):
    i = plsc.bitcast(x, jnp.int32)
    i = 0x5F3759DF - (i >> 1)
    y = plsc.bitcast(i, jnp.float32)
    for _ in range(3):
        y = y * (1.5 - 0.5 * x * y * y)
    return y


def _tree_sum(xs):
    xs = list(xs)
    while len(xs) > 1:
        nxt = [xs[i] + xs[i + 1] for i in range(0, len(xs) - 1, 2)]
        if len(xs) % 2:
            nxt.append(xs[-1])
        xs = nxt
    return xs[0]


def _body(emb_hbm, seg_hbm, edges_hbm, w_hbm, out_hbm,
          emb_v, seg_v, tab_v, e0_v, e1_v, w_v, idxa_v, idxb_v,
          acc_v, row_v, shared, sem_in):
    cid = lax.axis_index("c")
    sid = lax.axis_index("s")
    wid = cid * NS + sid

    pix = emb_v.shape[1]
    ngrp = pix // L
    ept = e0_v.shape[0]

    iota = lax.iota(jnp.int32, L)
    zeros = jnp.zeros((L,), jnp.float32)
    ones = jnp.ones((L,), jnp.float32)
    cols = [jnp.full((L,), d, jnp.int32) for d in range(D)]

    dins = [
        pltpu.make_async_copy(emb_hbm.at[cid, :, pl.ds(sid * pix, pix)],
                              emb_v, sem_in),
        pltpu.make_async_copy(seg_hbm.at[cid, sid], seg_v, sem_in),
        pltpu.make_async_copy(edges_hbm.at[cid, 0, pl.ds(sid * ept, ept)],
                              e0_v, sem_in),
        pltpu.make_async_copy(edges_hbm.at[cid, 1, pl.ds(sid * ept, ept)],
                              e1_v, sem_in),
        pltpu.make_async_copy(w_hbm.at[cid, pl.ds(sid * ept, ept)], w_v,
                              sem_in),
    ]
    for dsc in dins:
        dsc.start()

    for r in range(ROWS):
        tab_v[r] = zeros
    for r in range(C // L):
        idxa_v[pl.ds(r * L, L)] = iota + r * L
    idxb_v[...] = iota + C
    acc_v[...] = zeros

    @pl.when(sid == 0)
    def _():
        pltpu.sync_copy(tab_v, shared)
    plsc.subcore_barrier()

    for dsc in dins:
        dsc.wait()

    @pl.loop(0, ngrp)
    def _(g):
        s16 = seg_v[g >> 3, pl.ds((g & 7) * L, L)]
        es = [emb_v[d, pl.ds(g * L, L)] for d in range(D)]
        for d in range(D):
            plsc.addupdate_scatter(tab_v, [s16, cols[d]], es[d])
        plsc.addupdate_scatter(tab_v, [C + (s16 >> 4), s16 & (L - 1)], ones)

    pltpu.sync_copy(tab_v.at[pl.ds(0, C)], shared.at[idxa_v], add=True)
    pltpu.sync_copy(tab_v.at[pl.ds(C, L)], shared.at[idxb_v], add=True)
    plsc.subcore_barrier()

    pltpu.sync_copy(shared, tab_v)
    for grp in range(C // L):
        rows = iota + grp * L
        vs = [plsc.load_gather(tab_v, [rows, cols[d]]) for d in range(D)]
        nsq = _tree_sum([v * v for v in vs])
        rs = _rsqrt(jnp.maximum(nsq, 1e-20))
        for d in range(D):
            plsc.store_scatter(tab_v, [rows, cols[d]], vs[d] * rs)

    @pl.loop(0, ngrp)
    def _(g):
        s16 = seg_v[g >> 3, pl.ds((g & 7) * L, L)]
        es = [emb_v[d, pl.ds(g * L, L)] for d in range(D)]
        ms = [plsc.load_gather(tab_v, [s16, cols[d]]) for d in range(D)]
        n16 = plsc.load_gather(tab_v, [C + (s16 >> 4), s16 & (L - 1)])
        dot = _tree_sum([e * m for e, m in zip(es, ms)])
        term = jnp.maximum((1.0 - DELTA_VAR) - dot, 0.0) / n16
        acc_v[...] = acc_v[...] + term

    inter = zeros
    for k in range(ept // L):
        a = e0_v[pl.ds(k * L, L)]
        b = e1_v[pl.ds(k * L, L)]
        mas = [plsc.load_gather(tab_v, [a, cols[d]]) for d in range(D)]
        mbs = [plsc.load_gather(tab_v, [b, cols[d]]) for d in range(D)]
        dd = _tree_sum([x * y for x, y in zip(mas, mbs)])
        wk = w_v[pl.ds(k * L, L)]
        inter = inter + jnp.maximum(DELTA_DIST - wk * (1.0 - dd), 0.0)

    maxc = jnp.full((L,), -1, jnp.int32)
    for r in range(C // L):
        cv = tab_v[C + r]
        maxc = jnp.maximum(maxc, jnp.where(cv > 0.0, iota + r * L, -1))
    c_div = jnp.broadcast_to(jnp.max(maxc) + 1, (L,)).astype(jnp.float32)

    inv_e = 1.0 / float(ept * NS)
    row_v[...] = BETA * (acc_v[...] / c_div) + (ALPHA * inv_e) * inter
    pltpu.sync_copy(row_v, out_hbm.at[wid])


@jax.jit
def _run(emb, seg, edges, w):
    b, d, hw = emb.shape
    pix = hw // NS
    ept = edges.shape[2] // NS
    kern = pl.kernel(
        _body,
        out_type=jax.ShapeDtypeStruct((NC * NS, L), jnp.float32),
        mesh=plsc.VectorSubcoreMesh(core_axis_name="c", subcore_axis_name="s"),
        compiler_params=pltpu.CompilerParams(
            needs_layout_passes=False, use_tc_tiling_on_sc=False),
        scratch_types=[
            pltpu.VMEM((D, pix), jnp.float32),
            pltpu.VMEM((pix // C, C), jnp.int32),
            pltpu.VMEM((ROWS, L), jnp.float32),
            pltpu.VMEM((ept,), jnp.int32),
            pltpu.VMEM((ept,), jnp.int32),
            pltpu.VMEM((ept,), jnp.float32),
            pltpu.VMEM((C,), jnp.int32),
            pltpu.VMEM((L,), jnp.int32),
            pltpu.VMEM((L,), jnp.float32),
            pltpu.VMEM((L,), jnp.float32),
            pltpu.VMEM_SHARED((ROWS, L), jnp.float32),
            pltpu.SemaphoreType.DMA,
        ],
    )
    out = kern(emb, seg, edges, w)
    return jnp.sum(out)


def kernel(embeddings, sp_seg, edges, weights, chunks=4):
    b, d, h, w = embeddings.shape
    hw = h * w
    emb = embeddings.reshape(b, d, hw)
    seg = sp_seg.reshape(b, hw).astype(jnp.int32).reshape(b, NS, hw // NS // C, C)
    return _run(emb, seg, edges.astype(jnp.int32), weights)
